# Initial kernel scaffold; baseline (speedup 1.0000x reference)
import jax
import jax.numpy as jnp
from jax.experimental import pallas as pl


def kernel(input, adj, W1, b1, W2, b2):
    raise NotImplementedError("write your pallas kernel here")


# speedup vs baseline: 1.1877x; 1.1877x over previous
import functools

import jax
import jax.numpy as jnp
from jax import lax
from jax.experimental import pallas as pl
from jax.experimental.pallas import tpu as pltpu
from jax.experimental.pallas import tpu_sc as plsc

N = 10000
K = 32
C = 128
NC = 2
NS = 16
NW = NC * NS
NP = 10240
NPW = NP // NW
CH = 4
NCHUNK = NPW // CH


def _sc_body(xt_hbm, adjf_hbm, out_hbm, idx_v, rows_v, obuf_v, sem):
    wid = lax.axis_index("s") * NC + lax.axis_index("c")
    base = wid * NPW

    @pl.loop(0, NCHUNK)
    def _chunk(c):
        node0 = base + c * CH
        pltpu.sync_copy(adjf_hbm.at[pl.ds(node0 * K, CH * K)], idx_v)
        pltpu.async_copy(xt_hbm.at[idx_v], rows_v, sem).wait()
        for i in range(CH):
            for g in range(C // 16):
                acc = rows_v[i * K, pl.ds(g * 16, 16)]
                for k in range(1, K):
                    acc = acc + rows_v[i * K + k, pl.ds(g * 16, 16)]
                obuf_v[i, pl.ds(g * 16, 16)] = acc
        pltpu.sync_copy(obuf_v, out_hbm.at[pl.ds(node0, CH)])


_sc_gather_sum = functools.partial(
    pl.kernel,
    out_type=jax.ShapeDtypeStruct((NP, C), jnp.float32),
    mesh=plsc.VectorSubcoreMesh(core_axis_name="c", subcore_axis_name="s"),
    scratch_types=[
        pltpu.VMEM((CH * K,), jnp.int32),
        pltpu.VMEM((CH * K, C), jnp.float32),
        pltpu.VMEM((CH, C), jnp.float32),
        pltpu.SemaphoreType.DMA,
    ],
)(_sc_body)


def _tc_body(x_ref, g_ref, w1_ref, w2_ref, b_ref, o_ref):
    o_ref[...] = (
        jnp.dot(w1_ref[...], x_ref[...], preferred_element_type=jnp.float32)
        + lax.dot_general(
            w2_ref[...], g_ref[...],
            (((1,), (1,)), ((), ())),
            preferred_element_type=jnp.float32,
        )
        + b_ref[...]
    )


NB = 1024


def _tc_matmuls(x, gsum, w1, w2s, bias2d):
    grid = (NP // NB,)
    return pl.pallas_call(
        _tc_body,
        grid=grid,
        in_specs=[
            pl.BlockSpec((C, NB), lambda i: (0, i)),
            pl.BlockSpec((NB, C), lambda i: (i, 0)),
            pl.BlockSpec((C, C), lambda i: (0, 0)),
            pl.BlockSpec((C, C), lambda i: (0, 0)),
            pl.BlockSpec((C, 1), lambda i: (0, 0)),
        ],
        out_specs=pl.BlockSpec((C, NB), lambda i: (0, i)),
        out_shape=jax.ShapeDtypeStruct((C, N), jnp.float32),
    )(x, gsum, w1, w2s, bias2d)


def kernel(input, adj, W1, b1, W2, b2):
    x = input.reshape(C, N)
    xt = x.T
    adjf = jnp.pad(adj.astype(jnp.int32).reshape(-1), (0, (NP - N) * K))
    gsum = _sc_gather_sum(xt, adjf)
    w2s = W2 * (1.0 / K)
    bias2d = (b1 + b2)[:, None]
    out = _tc_matmuls(x, gsum, W1, w2s, bias2d)
    return out.reshape(1, C, N)


# speedup vs baseline: 1.5703x; 1.3221x over previous
import functools

import jax
import jax.numpy as jnp
from jax import lax
from jax.experimental import pallas as pl
from jax.experimental.pallas import tpu as pltpu
from jax.experimental.pallas import tpu_sc as plsc

N = 10000
K = 32
C = 128
NC = 2
NS = 16
NW = NC * NS
NP = 10240
NPW = NP // NW
CH = 4
NCHUNK = NPW // CH


NBUF = 2


def _sc_body(xt_hbm, adjf_hbm, out_hbm, idx_all, rows, obuf_v, gsem):
    wid = lax.axis_index("s") * NC + lax.axis_index("c")
    base = wid * NPW
    pltpu.sync_copy(adjf_hbm.at[pl.ds(base * K, NPW * K)], idx_all)

    def _start(c, b):
        pltpu.async_copy(
            xt_hbm.at[idx_all.at[pl.ds(c * CH * K, CH * K)]], rows[b], gsem[b])

    def _wait(c, b):
        pltpu.make_async_copy(
            xt_hbm.at[idx_all.at[pl.ds(c * CH * K, CH * K)]], rows[b], gsem[b]
        ).wait()

    for b in range(NBUF):
        _start(b, b)

    @pl.loop(0, NCHUNK, step=NBUF)
    def _chunk(c):
        for b in range(NBUF):
            cc = c + b
            _wait(cc, b)
            for i in range(CH):
                for g in range(C // 16):
                    acc = rows[b][i * K, pl.ds(g * 16, 16)]
                    for k in range(1, K):
                        acc = acc + rows[b][i * K + k, pl.ds(g * 16, 16)]
                    obuf_v[i, pl.ds(g * 16, 16)] = acc
            pltpu.sync_copy(obuf_v, out_hbm.at[pl.ds(base + cc * CH, CH)])

            @pl.when(cc + NBUF < NCHUNK)
            def _():
                _start(cc + NBUF, b)


_sc_gather_sum = functools.partial(
    pl.kernel,
    out_type=jax.ShapeDtypeStruct((NP, C), jnp.float32),
    mesh=plsc.VectorSubcoreMesh(core_axis_name="c", subcore_axis_name="s"),
    scratch_types=[
        pltpu.VMEM((NPW * K,), jnp.int32),
        [pltpu.VMEM((CH * K, C), jnp.float32) for _ in range(NBUF)],
        pltpu.VMEM((CH, C), jnp.float32),
        [pltpu.SemaphoreType.DMA for _ in range(NBUF)],
    ],
)(_sc_body)


def _tc_body(x_ref, g_ref, w1_ref, w2_ref, b_ref, o_ref):
    o_ref[...] = (
        jnp.dot(w1_ref[...], x_ref[...], preferred_element_type=jnp.float32)
        + lax.dot_general(
            w2_ref[...], g_ref[...],
            (((1,), (1,)), ((), ())),
            preferred_element_type=jnp.float32,
        )
        + b_ref[...]
    )


NB = 1024


def _tc_matmuls(x, gsum, w1, w2s, bias2d):
    grid = (NP // NB,)
    return pl.pallas_call(
        _tc_body,
        grid=grid,
        in_specs=[
            pl.BlockSpec((C, NB), lambda i: (0, i)),
            pl.BlockSpec((NB, C), lambda i: (i, 0)),
            pl.BlockSpec((C, C), lambda i: (0, 0)),
            pl.BlockSpec((C, C), lambda i: (0, 0)),
            pl.BlockSpec((C, 1), lambda i: (0, 0)),
        ],
        out_specs=pl.BlockSpec((C, NB), lambda i: (0, i)),
        out_shape=jax.ShapeDtypeStruct((C, N), jnp.float32),
    )(x, gsum, w1, w2s, bias2d)


def kernel(input, adj, W1, b1, W2, b2):
    x = input.reshape(C, N)
    xt = x.T
    adjf = jnp.pad(adj.astype(jnp.int32).reshape(-1), (0, (NP - N) * K))
    gsum = _sc_gather_sum(xt, adjf)
    w2s = W2 * (1.0 / K)
    bias2d = (b1 + b2)[:, None]
    out = _tc_matmuls(x, gsum, W1, w2s, bias2d)
    return out.reshape(1, C, N)
